# baseline (speedup 1.0000x reference)
import jax
import jax.numpy as jnp
from jax.experimental import pallas as pl
from jax.experimental.pallas import tpu as pltpu

Y = 64
SUB, LANE = 256, 128
BB = 8


def kernel(tables, sigma, base_obs, actions, responses, t):
    B, V, N = tables.shape
    T = actions.shape[1]
    assert N == SUB * LANE and B % BB == 0

    actions = actions.astype(jnp.int32)
    responses = responses.astype(jnp.int32)
    base_obs = base_obs.astype(jnp.int32)

    active = jnp.arange(T, dtype=jnp.int32)[None, :] < t
    use_real = active & (actions != V)
    a_c = jnp.clip(actions, 0, V - 1)
    rows = jnp.concatenate(
        [jnp.zeros((B, 1), jnp.int32), jnp.where(use_real, a_c, 0)], axis=1
    )
    vals = jnp.concatenate(
        [base_obs[:, None], jnp.where(use_real, responses, base_obs[:, None])],
        axis=1,
    )

    BIG = jnp.int32(1 << 20)
    hit = rows[:, None, :] == jnp.arange(V, dtype=jnp.int32)[None, :, None]
    vmin = jnp.min(jnp.where(hit, vals[:, None, :], BIG), axis=2)
    vmax = jnp.max(jnp.where(hit, vals[:, None, :], -BIG), axis=2)
    con = jnp.any(hit, axis=2)
    req = jnp.where(con, vmin, -1).astype(jnp.int32)
    feasible = jnp.all(~con | (vmin == vmax), axis=1)
    ncon = jnp.where(
        feasible, jnp.sum(con.astype(jnp.int32), axis=1), V + 1
    ).astype(jnp.int32)

    s3 = sigma.reshape(B // BB, BB, SUB, LANE)
    req3 = req.reshape(B // BB, BB, V)[:, :, :, None]
    t4 = tables.reshape(B // BB, BB, V, N)

    grid_spec = pltpu.PrefetchScalarGridSpec(
        num_scalar_prefetch=1,
        grid=(B // BB,),
        in_specs=(
            pl.BlockSpec((1, BB, V, N), lambda g, s: (g, 0, 0, 0)),
            pl.BlockSpec((1, BB, V, 1), lambda g, s: (g, 0, 0, 0)),
            pl.BlockSpec((1, BB, SUB, LANE), lambda g, s: (g, 0, 0, 0)),
        ),
        out_specs=pl.BlockSpec((1, BB, 1, Y), lambda g, s: (g, 0, 0, 0)),
    )

    def body(ncon_ref, tab_ref, req_ref, sig_ref, out_ref):
        g = pl.program_id(0)
        for i in range(BB):
            tab = tab_ref[0, i]
            req = req_ref[0, i]
            eq = (tab == req).astype(jnp.float32)
            cnt = jax.lax.dot_general(
                jnp.ones((1, V), jnp.float32),
                eq,
                (((1,), (0,)), ((), ())),
                preferred_element_type=jnp.float32,
            )
            key = (
                cnt.reshape(SUB, LANE) * Y
                + sig_ref[0, i].astype(jnp.float32)
                - (ncon_ref[g * BB + i] * Y).astype(jnp.float32)
            )
            hist = jnp.stack(
                [jnp.sum((key == c).astype(jnp.float32)) for c in range(Y)]
            ).reshape(1, Y)
            z = jnp.maximum(jnp.sum(hist), 1.0)
            out_ref[0, i] = jnp.log(jnp.maximum(hist / z, 1e-9))

    out = pl.pallas_call(
        body,
        grid_spec=grid_spec,
        out_shape=jax.ShapeDtypeStruct((B // BB, BB, 1, Y), jnp.float32),
        compiler_params=pltpu.CompilerParams(
            dimension_semantics=("arbitrary",)
        ),
    )(ncon, t4, req3, s3)
    return out.reshape(B, Y)


# speedup vs baseline: 1.0408x; 1.0408x over previous
import jax
import jax.numpy as jnp
from jax.experimental import pallas as pl
from jax.experimental.pallas import tpu as pltpu

Y = 64
SUB, LANE = 256, 128
BB = 4


def kernel(tables, sigma, base_obs, actions, responses, t):
    B, V, N = tables.shape
    T = actions.shape[1]
    assert N == SUB * LANE and B % BB == 0

    actions = actions.astype(jnp.int32)
    responses = responses.astype(jnp.int32)
    base_obs = base_obs.astype(jnp.int32)

    active = jnp.arange(T, dtype=jnp.int32)[None, :] < t
    use_real = active & (actions != V)
    a_c = jnp.clip(actions, 0, V - 1)
    rows = jnp.concatenate(
        [jnp.zeros((B, 1), jnp.int32), jnp.where(use_real, a_c, 0)], axis=1
    )
    vals = jnp.concatenate(
        [base_obs[:, None], jnp.where(use_real, responses, base_obs[:, None])],
        axis=1,
    )

    BIG = jnp.int32(1 << 20)
    hit = rows[:, None, :] == jnp.arange(V, dtype=jnp.int32)[None, :, None]
    vmin = jnp.min(jnp.where(hit, vals[:, None, :], BIG), axis=2)
    vmax = jnp.max(jnp.where(hit, vals[:, None, :], -BIG), axis=2)
    con = jnp.any(hit, axis=2)
    req = jnp.where(con, vmin, -1).astype(jnp.int32)
    feasible = jnp.all(~con | (vmin == vmax), axis=1)
    ncon = jnp.where(
        feasible, jnp.sum(con.astype(jnp.int32), axis=1), V + 1
    ).astype(jnp.int32)

    s3 = sigma.reshape(B // BB, BB, SUB, LANE).astype(jnp.float32)
    req3 = req.reshape(B // BB, BB, V)[:, :, :, None]
    t4 = tables.reshape(B // BB, BB, V, N)

    grid_spec = pltpu.PrefetchScalarGridSpec(
        num_scalar_prefetch=1,
        grid=(B // BB,),
        in_specs=(
            pl.BlockSpec((1, BB, V, N), lambda g, s: (g, 0, 0, 0)),
            pl.BlockSpec((1, BB, V, 1), lambda g, s: (g, 0, 0, 0)),
            pl.BlockSpec((1, BB, SUB, LANE), lambda g, s: (g, 0, 0, 0)),
        ),
        out_specs=pl.BlockSpec((1, BB, 1, Y), lambda g, s: (g, 0, 0, 0)),
    )

    def body(ncon_ref, tab_ref, req_ref, sig_ref, out_ref):
        g = pl.program_id(0)
        for i in range(BB):
            tab = tab_ref[0, i]
            req = req_ref[0, i]
            eq = (tab == req).astype(jnp.float32)
            cnt64 = jax.lax.dot_general(
                jnp.full((1, V), float(Y), jnp.float32),
                eq,
                (((1,), (0,)), ((), ())),
                preferred_element_type=jnp.float32,
            )
            key = (
                cnt64.reshape(SUB, LANE)
                + sig_ref[0, i]
                - (ncon_ref[g * BB + i] * Y).astype(jnp.float32)
            )
            hist = jnp.stack(
                [jnp.sum((key == c).astype(jnp.float32)) for c in range(Y)]
            ).reshape(1, Y)
            z = jnp.maximum(jnp.sum(hist), 1.0)
            out_ref[0, i] = jnp.log(jnp.maximum(hist / z, 1e-9))

    out = pl.pallas_call(
        body,
        grid_spec=grid_spec,
        out_shape=jax.ShapeDtypeStruct((B // BB, BB, 1, Y), jnp.float32),
        compiler_params=pltpu.CompilerParams(
            dimension_semantics=("arbitrary",)
        ),
    )(ncon, t4, req3, s3)
    return out.reshape(B, Y)


# speedup vs baseline: 1.1176x; 1.0738x over previous
import jax
import jax.numpy as jnp
from jax.experimental import pallas as pl
from jax.experimental.pallas import tpu as pltpu

Y = 64
SUB, LANE = 256, 128
BB = 4


def kernel(tables, sigma, base_obs, actions, responses, t):
    B, V, N = tables.shape
    T = actions.shape[1]
    assert N == SUB * LANE and B % BB == 0

    actions = actions.astype(jnp.int32)
    responses = responses.astype(jnp.int32)
    base_obs = base_obs.astype(jnp.int32)

    active = jnp.arange(T, dtype=jnp.int32)[None, :] < t
    use_real = active & (actions != V)
    a_c = jnp.clip(actions, 0, V - 1)
    rows = jnp.concatenate(
        [jnp.zeros((B, 1), jnp.int32), jnp.where(use_real, a_c, 0)], axis=1
    )
    vals = jnp.concatenate(
        [base_obs[:, None], jnp.where(use_real, responses, base_obs[:, None])],
        axis=1,
    )

    BIG = jnp.int32(1 << 20)
    hit = rows[:, None, :] == jnp.arange(V, dtype=jnp.int32)[None, :, None]
    vmin = jnp.min(jnp.where(hit, vals[:, None, :], BIG), axis=2)
    vmax = jnp.max(jnp.where(hit, vals[:, None, :], -BIG), axis=2)
    con = jnp.any(hit, axis=2)
    req = jnp.where(con, vmin, -1).astype(jnp.int32)
    feasible = jnp.all(~con | (vmin == vmax), axis=1)
    ncon = jnp.where(
        feasible, jnp.sum(con.astype(jnp.int32), axis=1), V + 1
    ).astype(jnp.int32)

    s3 = sigma.reshape(B // BB, BB, SUB, LANE)
    req3 = req.reshape(B // BB, BB, V)[:, :, :, None]
    t4 = tables.reshape(B // BB, BB, V, N)

    grid_spec = pltpu.PrefetchScalarGridSpec(
        num_scalar_prefetch=1,
        grid=(B // BB,),
        in_specs=(
            pl.BlockSpec((1, BB, V, N), lambda g, s: (g, 0, 0, 0)),
            pl.BlockSpec((1, BB, V, 1), lambda g, s: (g, 0, 0, 0)),
            pl.BlockSpec((1, BB, SUB, LANE), lambda g, s: (g, 0, 0, 0)),
        ),
        out_specs=pl.BlockSpec((1, BB, 1, Y), lambda g, s: (g, 0, 0, 0)),
    )

    def body(ncon_ref, tab_ref, req_ref, sig_ref, out_ref):
        g = pl.program_id(0)
        for i in range(BB):
            tab = tab_ref[0, i]
            req = req_ref[0, i]
            eq = (tab == req).astype(jnp.float32)
            cnt64 = jax.lax.dot_general(
                jnp.full((1, V), float(Y), jnp.float32),
                eq,
                (((1,), (0,)), ((), ())),
                preferred_element_type=jnp.float32,
            )
            key = (
                cnt64.reshape(SUB, LANE)
                + sig_ref[0, i].astype(jnp.float32)
                - (ncon_ref[g * BB + i] * Y).astype(jnp.float32)
            )
            hist = jnp.stack(
                [jnp.sum((key == c).astype(jnp.float32)) for c in range(Y)]
            ).reshape(1, Y)
            z = jnp.maximum(jnp.sum(hist), 1.0)
            out_ref[0, i] = jnp.log(jnp.maximum(hist / z, 1e-9))

    out = pl.pallas_call(
        body,
        grid_spec=grid_spec,
        out_shape=jax.ShapeDtypeStruct((B // BB, BB, 1, Y), jnp.float32),
        compiler_params=pltpu.CompilerParams(
            dimension_semantics=("arbitrary",)
        ),
    )(ncon, t4, req3, s3)
    return out.reshape(B, Y)
